# Initial kernel scaffold; baseline (speedup 1.0000x reference)
import numpy as np
import jax
import jax.numpy as jnp
from jax import lax
from jax.experimental import pallas as pl
from jax.experimental.pallas import tpu as pltpu

K_PART = 10
M_PART = 16
AJR = 32
NUM_HEAD = 4
HID = 64
N = K_PART * M_PART
DEG = N - 1
HD2 = HID + 2 + K_PART
D_MODEL = NUM_HEAD * HD2

_NEG = jnp.float32(-3.4e38)


def _prep_kernel(x_ref, label_ref, h0_ref, src_ref, w_ref, et0_ref, et1_ref,
                 d_ref, pe_ref,
                 l0w, l0b, l1w, l1b, l2w, l2b, l3w, l3b, l4w, l4b, l5w, l5b,
                 m0w, m0b, m1w, m1b, m2w, m2b,
                 gs_ref,
                 h_out, hfull_out, sa_out, sb_out, vvl_out):
    f32 = jnp.float32
    x = x_ref[...]
    label = label_ref[...]
    src = src_ref[...]
    wv = w_ref[...]
    et0 = et0_ref[...]
    et1 = et1_ref[...]
    dv = d_ref[...]

    m1 = wv * et0
    m2 = wv * et1
    s1 = jnp.sum(m1, axis=1, keepdims=True)
    s2 = jnp.sum(m2, axis=1, keepdims=True)

    iota_k = lax.broadcasted_iota(jnp.int32, (N, 1, N), 2)
    P1 = jnp.zeros((N, N), f32)
    P2 = jnp.zeros((N, N), f32)
    CH = 8
    for c in range(0, DEG, CH):
        e = min(c + CH, DEG)
        blk = src[:, c:e]
        mask = (blk[:, :, None] == iota_k).astype(f32)
        P1 = P1 + jnp.sum(mask * m1[:, c:e, None], axis=1)
        P2 = P2 + jnp.sum(mask * m2[:, c:e, None], axis=1)

    def topk(v, k):
        out = jnp.zeros((N, k), f32)
        kio = lax.broadcasted_iota(jnp.int32, (1, k), 1)
        jio = lax.broadcasted_iota(jnp.int32, (N, DEG), 1)

        def step(t, carry):
            v, out = carry
            m = jnp.max(v, axis=1, keepdims=True)
            idx = jnp.argmax(v, axis=1)[:, None]
            v = jnp.where(jio == idx, _NEG, v)
            out = out + m * (kio == t).astype(f32)
            return v, out

        _, out = lax.fori_loop(0, k, step, (v, out))
        return out

    n1e = topk(dv * et0, AJR)
    n2e = topk(dv * et1, M_PART - 1)

    def linT(v, W, b):
        return lax.dot_general(v, W[...], (((1,), (1,)), ((), ())),
                               preferred_element_type=f32) + b[...]

    base = (linT(x, l0w, l0b) + linT(label, l1w, l1b)
            + linT(n1e, l4w, l4b) + linT(n2e, l5w, l5b)
            + l2b[...] + l3b[...])

    def gnn_body(_, h):
        n1v = jnp.dot(P1, h, preferred_element_type=f32) / s1
        n2v = jnp.dot(P2, h, preferred_element_type=f32) / s2
        return jnp.maximum(
            base
            + lax.dot_general(n1v, l2w[...], (((1,), (1,)), ((), ())),
                              preferred_element_type=f32)
            + lax.dot_general(n2v, l3w[...], (((1,), (1,)), ((), ())),
                              preferred_element_type=f32),
            0.0)

    h = lax.fori_loop(0, gs_ref[0], gnn_body, h0_ref[...])
    h_out[...] = h

    hfull = jnp.concatenate([h + pe_ref[...], x, label], axis=1)
    hfull_out[...] = hfull

    G = lax.dot_general(hfull, label, (((0,), (0,)), ((), ())),
                        preferred_element_type=f32) / M_PART
    lgc = lax.dot_general(label, G, (((1,), (1,)), ((), ())),
                          preferred_element_type=f32)

    W0 = m0w[...]
    A1 = (lax.dot_general(hfull, W0[:, :HD2], (((1,), (1,)), ((), ())),
                          preferred_element_type=f32)
          + lax.dot_general(lgc, W0[:, 2 * HD2:3 * HD2], (((1,), (1,)), ((), ())),
                            preferred_element_type=f32)
          + m0b[...])
    B1 = (lax.dot_general(hfull, W0[:, HD2:2 * HD2], (((1,), (1,)), ((), ())),
                          preferred_element_type=f32)
          + lax.dot_general(lgc, W0[:, 3 * HD2:], (((1,), (1,)), ((), ())),
                            preferred_element_type=f32))

    W1 = m1w[...]
    W1s = W1[:, :HD2] + W1[:, HD2:2 * HD2] + W1[:, 2 * HD2:3 * HD2] + W1[:, 3 * HD2:]
    kk = lax.dot_general(hfull, W1s, (((1,), (1,)), ((), ())),
                         preferred_element_type=f32) + m1b[...]
    W2 = m2w[...]
    W2s = W2[:, :HD2] + W2[:, HD2:2 * HD2] + W2[:, 2 * HD2:3 * HD2] + W2[:, 3 * HD2:]
    vvl = lax.dot_general(hfull, W2s, (((1,), (1,)), ((), ())),
                          preferred_element_type=f32) + m2b[...]
    vvl_out[...] = vvl

    scale = jnp.float32(1.0 / np.sqrt(HD2))
    for hh in range(NUM_HEAD):
        sl = slice(hh * HD2, (hh + 1) * HD2)
        sa_out[hh] = lax.dot_general(A1[:, sl], kk[:, sl],
                                     (((1,), (1,)), ((), ())),
                                     preferred_element_type=f32) * scale
        sb_out[hh] = lax.dot_general(B1[:, sl], kk[:, sl],
                                     (((1,), (1,)), ((), ())),
                                     preferred_element_type=f32) * scale


_BI = 8


def _attn_kernel(sa_ref, sb_ref, vvl_ref, m3w, m3b, v1w, v1b, v2w, v2b,
                 s_out, q_out):
    f32 = jnp.float32
    i0 = pl.program_id(0) * _BI
    xo_parts = []
    for hh in range(NUM_HEAD):
        sa = sa_ref[hh]
        sb = sb_ref[hh]
        sab = sa[pl.ds(i0, _BI), :]
        sbb = sb[pl.ds(i0, _BI), :]

        def soft(qrows, krows):
            s = qrows[:, None, :] + krows[None, :, :]
            m = jnp.max(s, axis=-1, keepdims=True)
            p = jnp.exp(s - m)
            return p / jnp.sum(p, axis=-1, keepdims=True)

        psum = soft(sab, sb) + soft(sbb, sa)
        vv_h = vvl_ref[...][:, hh * HD2:(hh + 1) * HD2]
        xo = jnp.dot(psum.reshape(_BI * N, N), vv_h,
                     preferred_element_type=f32)
        xo_parts.append(xo)
    xo = jnp.concatenate(xo_parts, axis=1)
    S = lax.dot_general(xo, m3w[...], (((1,), (1,)), ((), ())),
                        preferred_element_type=f32) + 2.0 * m3b[...]
    s_out[...] = S
    r = jnp.maximum(
        lax.dot_general(S, v1w[...], (((1,), (1,)), ((), ())),
                        preferred_element_type=f32) + v1b[...], 0.0)
    q = lax.dot_general(r, v2w[...], (((1,), (1,)), ((), ())),
                        preferred_element_type=f32) + v2b[...]
    q_out[...] = q.reshape(_BI, N)


def _pe_table():
    dm = HID
    pos = np.arange(50)[:, None].astype(np.float32)
    div = np.exp(np.arange(0, dm, 2).astype(np.float32) * -(np.log(10000.0) / dm))
    pe = np.zeros((50, dm), dtype=np.float32)
    pe[:, 0::2] = np.sin(pos * div)
    pe[:, 1::2] = np.cos(pos * div)
    return pe


_PE = _pe_table()


def kernel(x, label, h0, src, w, e_type, d,
           l0_W, l0_b, l1_W, l1_b, l2_W, l2_b, l3_W, l3_b, l4_W, l4_b,
           l5_W, l5_b,
           mha0_W, mha0_b, mha1_W, mha1_b, mha2_W, mha2_b, mha3_W, mha3_b,
           v1_W, v1_b, v2_W, v2_b, gnn_step, max_step, remain_step):
    f32 = jnp.float32
    src = src.astype(jnp.int32)
    w2 = w[:, :, 0]
    et0 = e_type[:, :, 0]
    et1 = e_type[:, :, 1]
    d2 = d[:, :, 0]
    pe_row = jnp.asarray(_PE)[remain_step + 0 * max_step][None, :]
    gs = jnp.asarray(gnn_step, jnp.int32).reshape(1)

    row = lambda b: jnp.asarray(b, f32).reshape(1, -1)

    vmem = pl.BlockSpec(memory_space=pltpu.VMEM)
    n_in = 28
    h, hfull, SA, SB, vvl = pl.pallas_call(
        _prep_kernel,
        out_shape=(
            jax.ShapeDtypeStruct((N, HID), f32),
            jax.ShapeDtypeStruct((N, HD2), f32),
            jax.ShapeDtypeStruct((NUM_HEAD, N, N), f32),
            jax.ShapeDtypeStruct((NUM_HEAD, N, N), f32),
            jax.ShapeDtypeStruct((N, D_MODEL), f32),
        ),
        in_specs=[vmem] * (n_in - 1) + [pl.BlockSpec(memory_space=pltpu.SMEM)],
        out_specs=(vmem, vmem, vmem, vmem, vmem),
    )(x, label, h0, src, w2, et0, et1, d2, pe_row,
      l0_W, row(l0_b), l1_W, row(l1_b), l2_W, row(l2_b), l3_W, row(l3_b),
      l4_W, row(l4_b), l5_W, row(l5_b),
      mha0_W, row(mha0_b), mha1_W, row(mha1_b), mha2_W, row(mha2_b), gs)

    grid = N // _BI
    full = lambda shape: pl.BlockSpec(shape, lambda i: (0,) * len(shape))
    S2, Qs = pl.pallas_call(
        _attn_kernel,
        grid=(grid,),
        in_specs=[
            full((NUM_HEAD, N, N)),
            full((NUM_HEAD, N, N)),
            full((N, D_MODEL)),
            full((D_MODEL, D_MODEL)),
            full((1, D_MODEL)),
            full((HID // 2, D_MODEL)),
            full((1, HID // 2)),
            full((1, HID // 2)),
            full((1, 1)),
        ],
        out_specs=(
            pl.BlockSpec((_BI * N, D_MODEL), lambda i: (i, 0)),
            pl.BlockSpec((_BI, N), lambda i: (i, 0)),
        ),
        out_shape=(
            jax.ShapeDtypeStruct((N * N, D_MODEL), f32),
            jax.ShapeDtypeStruct((N, N), f32),
        ),
    )(SA, SB, vvl, mha3_W, row(mha3_b), v1_W, row(v1_b), row(v2_W),
      jnp.asarray(v2_b, f32).reshape(1, 1))

    S = S2.reshape(N * N, 1, D_MODEL)
    Q_sa = Qs.reshape(N * N)
    return (S, h, hfull, Q_sa)


# speedup vs baseline: 2.5437x; 2.5437x over previous
import numpy as np
import jax
import jax.numpy as jnp
from jax import lax
from jax.experimental import pallas as pl
from jax.experimental.pallas import tpu as pltpu

K_PART = 10
M_PART = 16
AJR = 32
NUM_HEAD = 4
HID = 64
N = K_PART * M_PART
DEG = N - 1
HD2 = HID + 2 + K_PART
D_MODEL = NUM_HEAD * HD2

_NEG = np.float32(-3.4e38)


def _prep_kernel(x_ref, label_ref, h0_ref, src_ref, w_ref, et0_ref, et1_ref,
                 d_ref, pe_ref,
                 l0w, l0b, l1w, l1b, l2w, l2b, l3w, l3b, l4w, l4b, l5w, l5b,
                 m0w, m0b, m1w, m1b, m2w, m2b,
                 gs_ref,
                 h_out, hfull_out, sa_out, sb_out, vvl_out):
    f32 = jnp.float32
    x = x_ref[...]
    label = label_ref[...]
    src = src_ref[...]
    wv = w_ref[...]
    et0 = et0_ref[...]
    et1 = et1_ref[...]
    dv = d_ref[...]

    m1 = wv * et0
    m2 = wv * et1
    s1 = jnp.sum(m1, axis=1, keepdims=True)
    s2 = jnp.sum(m2, axis=1, keepdims=True)

    iota_k = lax.broadcasted_iota(jnp.int32, (N, 1, N), 2)
    P1 = jnp.zeros((N, N), f32)
    P2 = jnp.zeros((N, N), f32)
    CH = 8
    for c in range(0, DEG, CH):
        e = min(c + CH, DEG)
        blk = src[:, c:e]
        mask = (blk[:, :, None] == iota_k).astype(f32)
        P1 = P1 + jnp.sum(mask * m1[:, c:e, None], axis=1)
        P2 = P2 + jnp.sum(mask * m2[:, c:e, None], axis=1)

    def topk(v, k):
        out = jnp.zeros((N, k), f32)
        kio = lax.broadcasted_iota(jnp.int32, (1, k), 1)
        jio = lax.broadcasted_iota(jnp.int32, (N, DEG), 1)

        def step(t, carry):
            v, out = carry
            m = jnp.max(v, axis=1, keepdims=True)
            idx = jnp.argmax(v, axis=1)[:, None]
            v = jnp.where(jio == idx, _NEG, v)
            out = out + m * (kio == t).astype(f32)
            return v, out

        _, out = lax.fori_loop(0, k, step, (v, out))
        return out

    n1e = topk(dv * et0, AJR)
    n2e = topk(dv * et1, M_PART - 1)

    def linT(v, W, b):
        return lax.dot_general(v, W[...], (((1,), (1,)), ((), ())),
                               preferred_element_type=f32) + b[...]

    base = (linT(x, l0w, l0b) + linT(label, l1w, l1b)
            + linT(n1e, l4w, l4b) + linT(n2e, l5w, l5b)
            + l2b[...] + l3b[...])

    def gnn_body(_, h):
        n1v = jnp.dot(P1, h, preferred_element_type=f32) / s1
        n2v = jnp.dot(P2, h, preferred_element_type=f32) / s2
        return jnp.maximum(
            base
            + lax.dot_general(n1v, l2w[...], (((1,), (1,)), ((), ())),
                              preferred_element_type=f32)
            + lax.dot_general(n2v, l3w[...], (((1,), (1,)), ((), ())),
                              preferred_element_type=f32),
            0.0)

    h = lax.fori_loop(0, gs_ref[0], gnn_body, h0_ref[...])
    h_out[...] = h

    hfull = jnp.concatenate([h + pe_ref[...], x, label], axis=1)
    hfull_out[...] = hfull

    G = lax.dot_general(hfull, label, (((0,), (0,)), ((), ())),
                        preferred_element_type=f32) / M_PART
    lgc = lax.dot_general(label, G, (((1,), (1,)), ((), ())),
                          preferred_element_type=f32)

    W0 = m0w[...]
    A1 = (lax.dot_general(hfull, W0[:, :HD2], (((1,), (1,)), ((), ())),
                          preferred_element_type=f32)
          + lax.dot_general(lgc, W0[:, 2 * HD2:3 * HD2], (((1,), (1,)), ((), ())),
                            preferred_element_type=f32)
          + m0b[...])
    B1 = (lax.dot_general(hfull, W0[:, HD2:2 * HD2], (((1,), (1,)), ((), ())),
                          preferred_element_type=f32)
          + lax.dot_general(lgc, W0[:, 3 * HD2:], (((1,), (1,)), ((), ())),
                            preferred_element_type=f32))

    W1 = m1w[...]
    W1s = W1[:, :HD2] + W1[:, HD2:2 * HD2] + W1[:, 2 * HD2:3 * HD2] + W1[:, 3 * HD2:]
    kk = lax.dot_general(hfull, W1s, (((1,), (1,)), ((), ())),
                         preferred_element_type=f32) + m1b[...]
    W2 = m2w[...]
    W2s = W2[:, :HD2] + W2[:, HD2:2 * HD2] + W2[:, 2 * HD2:3 * HD2] + W2[:, 3 * HD2:]
    vvl = lax.dot_general(hfull, W2s, (((1,), (1,)), ((), ())),
                          preferred_element_type=f32) + m2b[...]
    vvl_out[...] = vvl

    scale = np.float32(1.0 / np.sqrt(HD2))
    for hh in range(NUM_HEAD):
        sl = slice(hh * HD2, (hh + 1) * HD2)
        sa_out[hh] = lax.dot_general(A1[:, sl], kk[:, sl],
                                     (((1,), (1,)), ((), ())),
                                     preferred_element_type=f32) * scale
        sb_out[hh] = lax.dot_general(B1[:, sl], kk[:, sl],
                                     (((1,), (1,)), ((), ())),
                                     preferred_element_type=f32) * scale


_BI = 8


def _attn_kernel(sa_ref, sb_ref, vvl_ref, m3w, m3b, v1w, v1b, v2w, v2b,
                 s_out, q_out):
    f32 = jnp.float32
    i0 = pl.program_id(0) * _BI
    vvl = vvl_ref[...]

    def soft(row, mat):
        s = row + mat
        m = jnp.max(s, axis=-1, keepdims=True)
        p = jnp.exp(s - m)
        return p / jnp.sum(p, axis=-1, keepdims=True)

    for li in range(_BI):
        xo_parts = []
        for hh in range(NUM_HEAD):
            sa = sa_ref[hh]
            sb = sb_ref[hh]
            sa_i = sa_ref[hh, pl.ds(i0 + li, 1), :]
            sb_i = sb_ref[hh, pl.ds(i0 + li, 1), :]
            psum = soft(sa_i, sb) + soft(sb_i, sa)
            xo_parts.append(
                jnp.dot(psum, vvl[:, hh * HD2:(hh + 1) * HD2],
                        preferred_element_type=f32))
        xo = jnp.concatenate(xo_parts, axis=1)
        S = lax.dot_general(xo, m3w[...], (((1,), (1,)), ((), ())),
                            preferred_element_type=f32) + 2.0 * m3b[...]
        s_out[li * N:(li + 1) * N, :] = S
        r = jnp.maximum(
            lax.dot_general(S, v1w[...], (((1,), (1,)), ((), ())),
                            preferred_element_type=f32) + v1b[...], 0.0)
        q = jnp.sum(r * v2w[...], axis=1, keepdims=True) + v2b[0, 0]
        q_out[li * N:(li + 1) * N, :] = q


def _pe_table():
    dm = HID
    pos = np.arange(50)[:, None].astype(np.float32)
    div = np.exp(np.arange(0, dm, 2).astype(np.float32) * -(np.log(10000.0) / dm))
    pe = np.zeros((50, dm), dtype=np.float32)
    pe[:, 0::2] = np.sin(pos * div)
    pe[:, 1::2] = np.cos(pos * div)
    return pe


_PE = _pe_table()


def kernel(x, label, h0, src, w, e_type, d,
           l0_W, l0_b, l1_W, l1_b, l2_W, l2_b, l3_W, l3_b, l4_W, l4_b,
           l5_W, l5_b,
           mha0_W, mha0_b, mha1_W, mha1_b, mha2_W, mha2_b, mha3_W, mha3_b,
           v1_W, v1_b, v2_W, v2_b, gnn_step, max_step, remain_step):
    f32 = jnp.float32
    src = src.astype(jnp.int32)
    w2 = w[:, :, 0]
    et0 = e_type[:, :, 0]
    et1 = e_type[:, :, 1]
    d2 = d[:, :, 0]
    pe_row = jnp.asarray(_PE)[remain_step + 0 * max_step][None, :]
    gs = jnp.asarray(gnn_step, jnp.int32).reshape(1)

    row = lambda b: jnp.asarray(b, f32).reshape(1, -1)

    vmem = pl.BlockSpec(memory_space=pltpu.VMEM)
    n_in = 28
    h, hfull, SA, SB, vvl = pl.pallas_call(
        _prep_kernel,
        out_shape=(
            jax.ShapeDtypeStruct((N, HID), f32),
            jax.ShapeDtypeStruct((N, HD2), f32),
            jax.ShapeDtypeStruct((NUM_HEAD, N, N), f32),
            jax.ShapeDtypeStruct((NUM_HEAD, N, N), f32),
            jax.ShapeDtypeStruct((N, D_MODEL), f32),
        ),
        in_specs=[vmem] * (n_in - 1) + [pl.BlockSpec(memory_space=pltpu.SMEM)],
        out_specs=(vmem, vmem, vmem, vmem, vmem),
    )(x, label, h0, src, w2, et0, et1, d2, pe_row,
      l0_W, row(l0_b), l1_W, row(l1_b), l2_W, row(l2_b), l3_W, row(l3_b),
      l4_W, row(l4_b), l5_W, row(l5_b),
      mha0_W, row(mha0_b), mha1_W, row(mha1_b), mha2_W, row(mha2_b), gs)

    grid = N // _BI
    full = lambda shape: pl.BlockSpec(shape, lambda i: (0,) * len(shape))
    S2, Qs = pl.pallas_call(
        _attn_kernel,
        grid=(grid,),
        in_specs=[
            full((NUM_HEAD, N, N)),
            full((NUM_HEAD, N, N)),
            full((N, D_MODEL)),
            full((D_MODEL, D_MODEL)),
            full((1, D_MODEL)),
            full((HID // 2, D_MODEL)),
            full((1, HID // 2)),
            full((1, HID // 2)),
            full((1, 1)),
        ],
        out_specs=(
            pl.BlockSpec((_BI * N, D_MODEL), lambda i: (i, 0)),
            pl.BlockSpec((_BI * N, 1), lambda i: (i, 0)),
        ),
        out_shape=(
            jax.ShapeDtypeStruct((N * N, D_MODEL), f32),
            jax.ShapeDtypeStruct((N * N, 1), f32),
        ),
    )(SA, SB, vvl, mha3_W, row(mha3_b), v1_W, row(v1_b), row(v2_W),
      jnp.asarray(v2_b, f32).reshape(1, 1))

    S = S2.reshape(N * N, 1, D_MODEL)
    Q_sa = Qs.reshape(N * N)
    return (S, h, hfull, Q_sa)
